# Initial kernel scaffold; baseline (speedup 1.0000x reference)
import jax
import jax.numpy as jnp
from jax.experimental import pallas as pl


def kernel(words_seq, table):
    raise NotImplementedError("write your pallas kernel here")


# speedup vs baseline: 3.5819x; 3.5819x over previous
import functools

import jax
import jax.numpy as jnp
from jax import lax
from jax.experimental import pallas as pl
from jax.experimental.pallas import tpu as pltpu
from jax.experimental.pallas import tpu_sc as plsc

EMBED = 64
NC, NS = 2, 16
NW = NC * NS

B_TOTAL = 4096 * 200
B_PER_W = B_TOTAL // NW
GATHER = 128
K = 8
CHUNK = K * GATHER
NCHUNK = B_PER_W // CHUNK
ROWS_PER_W = B_PER_W // GATHER


def _sc_embed(idx2d, table):
    mesh = plsc.VectorSubcoreMesh(core_axis_name="c", subcore_axis_name="s")

    @functools.partial(
        pl.kernel,
        mesh=mesh,
        out_type=jax.ShapeDtypeStruct((B_TOTAL, EMBED), jnp.float32),
        scratch_types=[
            pltpu.VMEM((K, GATHER), jnp.int32),
            pltpu.VMEM((CHUNK, EMBED), jnp.float32),
            pltpu.SemaphoreType.DMA,
        ],
        compiler_params=pltpu.CompilerParams(use_tc_tiling_on_sc=False),
    )
    def body(idx_hbm, table_hbm, out_hbm, idx_v, rows_v, sem):
        wid = lax.axis_index("s") * NC + lax.axis_index("c")
        row_base = wid * ROWS_PER_W
        out_base = wid * B_PER_W

        def step(g, carry):
            pltpu.sync_copy(idx_hbm.at[pl.ds(row_base + g * K, K)], idx_v)
            copies = []
            for j in range(K):
                copies.append(
                    pltpu.async_copy(
                        table_hbm.at[idx_v.at[j]],
                        rows_v.at[pl.ds(j * GATHER, GATHER)],
                        sem,
                    )
                )
            for c in copies:
                c.wait()
            pltpu.sync_copy(rows_v, out_hbm.at[pl.ds(out_base + g * CHUNK, CHUNK)])
            return carry

        lax.fori_loop(0, NCHUNK, step, 0)

    return body(idx2d, table)


def kernel(words_seq, table):
    idx2d = words_seq.astype(jnp.int32).reshape(B_TOTAL // GATHER, GATHER)
    out = _sc_embed(idx2d, table.astype(jnp.float32))
    return out.reshape(words_seq.shape[0], words_seq.shape[1], EMBED)
